# Initial kernel scaffold; baseline (speedup 1.0000x reference)
import jax
import jax.numpy as jnp
from jax.experimental import pallas as pl


def kernel(box_cls, box_regression, iou_pred, anchors):
    raise NotImplementedError("write your pallas kernel here")


# speedup vs baseline: 2.5321x; 2.5321x over previous
import math

import jax
import jax.numpy as jnp
from jax import lax
from jax.experimental import pallas as pl
from jax.experimental.pallas import tpu as pltpu

_C = 80
_HW = 20000
_TOPN = 1000
_PAD_N = 1024
_POST = 100
_PAD_P = 128
_IMG_W = 1600.0
_IMG_H = 800.0
_NMS_T = 0.6
_SIGMA = 0.025
_PRE_T = 0.05
_DWH_CAP = math.log(1000.0 / 16)


def _score_body(cls_ref, iou_ref, out_ref):
    p = jax.nn.sigmoid(cls_ref[...])
    pi = jax.nn.sigmoid(iou_ref[...])
    s = jnp.sqrt(p * pi)
    out_ref[...] = jnp.where(p > _PRE_T, s, -jnp.inf)


def _nms_body(preds_ref, anch_ref, labels_ref, valid_ref,
              boxes_ref, keep_ref, iou_scr):
    a0 = anch_ref[0:1, :]
    a1 = anch_ref[1:2, :]
    a2 = anch_ref[2:3, :]
    a3 = anch_ref[3:4, :]
    w = a2 - a0 + 1.0
    h = a3 - a1 + 1.0
    cx = (a2 + a0) * 0.5
    cy = (a3 + a1) * 0.5
    dx = preds_ref[0:1, :] * 0.1
    dy = preds_ref[1:2, :] * 0.1
    dw = jnp.minimum(preds_ref[2:3, :] * 0.2, _DWH_CAP)
    dh = jnp.minimum(preds_ref[3:4, :] * 0.2, _DWH_CAP)
    pcx = dx * w + cx
    pcy = dy * h + cy
    pw = jnp.exp(dw) * w
    ph = jnp.exp(dh) * h
    x1 = jnp.clip(pcx - 0.5 * (pw - 1.0), 0.0, _IMG_W - 1.0)
    y1 = jnp.clip(pcy - 0.5 * (ph - 1.0), 0.0, _IMG_H - 1.0)
    x2 = jnp.clip(pcx + 0.5 * (pw - 1.0), 0.0, _IMG_W - 1.0)
    y2 = jnp.clip(pcy + 0.5 * (ph - 1.0), 0.0, _IMG_H - 1.0)
    boxes_ref[0:1, :] = x1
    boxes_ref[1:2, :] = y1
    boxes_ref[2:3, :] = x2
    boxes_ref[3:4, :] = y2

    lane = lax.broadcasted_iota(jnp.int32, (1, _PAD_N), 1)
    real = lane < _TOPN
    neg = jnp.float32(-jnp.inf)
    mx = jnp.maximum(
        jnp.maximum(jnp.max(jnp.where(real, x1, neg)),
                    jnp.max(jnp.where(real, x2, neg))),
        jnp.maximum(jnp.max(jnp.where(real, y1, neg)),
                    jnp.max(jnp.where(real, y2, neg))))
    off = labels_ref[...] * (mx + 1.0)
    ox1 = x1 + off
    oy1 = y1 + off
    ox2 = x2 + off
    oy2 = y2 + off

    area = (ox2 - ox1 + 1.0) * (oy2 - oy1 + 1.0)
    cx1 = jnp.transpose(ox1)
    cy1 = jnp.transpose(oy1)
    cx2 = jnp.transpose(ox2)
    cy2 = jnp.transpose(oy2)
    iw = jnp.clip(jnp.minimum(cx2, ox2) - jnp.maximum(cx1, ox1) + 1.0, 0.0, None)
    ih = jnp.clip(jnp.minimum(cy2, oy2) - jnp.maximum(cy1, oy1) + 1.0, 0.0, None)
    inter = iw * ih
    iou_scr[...] = inter / (jnp.transpose(area) + area - inter)

    vrow = valid_ref[...]

    def body(i, keep):
        row = iou_scr[pl.ds(i, 1), :]
        lower = lane < i
        sup = jnp.sum(jnp.where(lower & (row > _NMS_T), keep, 0.0)) > 0.0
        vi = jnp.sum(jnp.where(lane == i, vrow, 0.0)) > 0.0
        newv = jnp.where(vi & jnp.logical_not(sup), 1.0, 0.0)
        return jnp.where(lane == i, newv, keep)

    keep = lax.fori_loop(0, _TOPN, body, jnp.zeros((1, _PAD_N), jnp.float32))
    keep_ref[...] = keep


def _vote_body(resb_ref, rlab_ref, rval_ref,
               boxes_ref, labels_ref, cscore_ref,
               voted_ref, den_ref):
    bx1 = boxes_ref[0:1, :]
    by1 = boxes_ref[1:2, :]
    bx2 = boxes_ref[2:3, :]
    by2 = boxes_ref[3:4, :]
    rx1 = jnp.transpose(resb_ref[0:1, :])
    ry1 = jnp.transpose(resb_ref[1:2, :])
    rx2 = jnp.transpose(resb_ref[2:3, :])
    ry2 = jnp.transpose(resb_ref[3:4, :])
    area_c = (bx2 - bx1 + 1.0) * (by2 - by1 + 1.0)
    area_r = (rx2 - rx1 + 1.0) * (ry2 - ry1 + 1.0)
    iw = jnp.clip(jnp.minimum(rx2, bx2) - jnp.maximum(rx1, bx1) + 1.0, 0.0, None)
    ih = jnp.clip(jnp.minimum(ry2, by2) - jnp.maximum(ry1, by1) + 1.0, 0.0, None)
    inter = iw * ih
    iou = inter / (area_r + area_c - inter)
    m = ((iou > 0.01)
         & (jnp.transpose(rlab_ref[...]) == labels_ref[...])
         & (jnp.transpose(rval_ref[...]) > 0.0))
    pis = jnp.exp(-(1.0 - iou) ** 2 / _SIGMA) * cscore_ref[...]
    pis = jnp.where(m, pis, 0.0)
    den = jnp.sum(pis, axis=1, keepdims=True)
    dsafe = jnp.where(den > 0.0, den, 1.0)
    vx1 = jnp.sum(pis * bx1, axis=1, keepdims=True) / dsafe
    vy1 = jnp.sum(pis * by1, axis=1, keepdims=True) / dsafe
    vx2 = jnp.sum(pis * bx2, axis=1, keepdims=True) / dsafe
    vy2 = jnp.sum(pis * by2, axis=1, keepdims=True) / dsafe
    voted_ref[...] = jnp.concatenate([vx1, vy1, vx2, vy2], axis=1)
    den_ref[...] = den


def kernel(box_cls, box_regression, iou_pred, anchors):
    cls2 = box_cls.reshape(_C, _HW)
    iou2 = iou_pred.reshape(1, _HW)

    blk = 2048
    grid = (_HW + blk - 1) // blk
    s_map = pl.pallas_call(
        _score_body,
        grid=(grid,),
        in_specs=[
            pl.BlockSpec((_C, blk), lambda i: (0, i)),
            pl.BlockSpec((1, blk), lambda i: (0, i)),
        ],
        out_specs=pl.BlockSpec((_C, blk), lambda i: (0, i)),
        out_shape=jax.ShapeDtypeStruct((_C, _HW), jnp.float32),
    )(cls2, iou2)

    top_scores, top_idx = lax.top_k(s_map.reshape(-1), _TOPN)
    loc = top_idx % _HW
    labels = top_idx // _HW + 1
    valid = jnp.isfinite(top_scores)

    reg_t = box_regression.reshape(4, _HW)[:, loc]
    anch_t = anchors.astype(jnp.float32).T[:, loc]
    padn = _PAD_N - _TOPN
    reg_p = jnp.pad(reg_t, ((0, 0), (0, padn)))
    anch_p = jnp.pad(anch_t, ((0, 0), (0, padn)))
    lab_p = jnp.pad(labels.astype(jnp.float32)[None, :], ((0, 0), (0, padn)))
    val_p = jnp.pad(valid.astype(jnp.float32)[None, :], ((0, 0), (0, padn)))

    boxes_t, keep_f = pl.pallas_call(
        _nms_body,
        out_shape=(
            jax.ShapeDtypeStruct((4, _PAD_N), jnp.float32),
            jax.ShapeDtypeStruct((1, _PAD_N), jnp.float32),
        ),
        scratch_shapes=[pltpu.VMEM((_PAD_N, _PAD_N), jnp.float32)],
    )(reg_p, anch_p, lab_p, val_p)

    keep = keep_f[0, :_TOPN] > 0.0
    s_kept = jnp.where(keep, top_scores, -jnp.inf)
    res_scores, res_idx = lax.top_k(s_kept, _POST)
    res_valid = jnp.isfinite(res_scores)
    res_labels = labels[res_idx]

    padp = _PAD_P - _POST
    resb_p = jnp.pad(boxes_t[:, res_idx], ((0, 0), (0, padp)))
    rlab_p = jnp.pad(res_labels.astype(jnp.float32)[None, :], ((0, 0), (0, padp)))
    rval_p = jnp.pad(res_valid.astype(jnp.float32)[None, :], ((0, 0), (0, padp)))
    cscore = jnp.where(valid, top_scores, 0.0)
    csc_p = jnp.pad(cscore[None, :], ((0, 0), (0, padn)))

    voted, den = pl.pallas_call(
        _vote_body,
        out_shape=(
            jax.ShapeDtypeStruct((_PAD_P, 4), jnp.float32),
            jax.ShapeDtypeStruct((_PAD_P, 1), jnp.float32),
        ),
    )(resb_p, rlab_p, rval_p, boxes_t, lab_p, csc_p)

    den100 = den[:_POST, 0]
    boxes100 = boxes_t[:, res_idx].T
    res_boxes = jnp.where((den100 > 0.0)[:, None], voted[:_POST], boxes100)
    out_scores = jnp.where(res_valid, res_scores, 0.0)
    out5 = jnp.concatenate([res_boxes, out_scores[:, None]], axis=1)
    return out5, res_labels


# speedup vs baseline: 2.5332x; 1.0004x over previous
import math

import jax
import jax.numpy as jnp
from jax import lax
from jax.experimental import pallas as pl
from jax.experimental.pallas import tpu as pltpu

_C = 80
_HW = 20000
_TOPN = 1000
_PAD_N = 1024
_POST = 100
_PAD_P = 128
_IMG_W = 1600.0
_IMG_H = 800.0
_NMS_T = 0.6
_SIGMA = 0.025
_PRE_T = 0.05
_DWH_CAP = math.log(1000.0 / 16)


def _score_body(cls_ref, iou_ref, out_ref):
    p = jax.nn.sigmoid(cls_ref[...])
    pi = jax.nn.sigmoid(iou_ref[...])
    s = jnp.sqrt(p * pi)
    out_ref[...] = jnp.where(p > _PRE_T, s, -jnp.inf)


def _nms_body(preds_ref, anch_ref, labels_ref, valid_ref,
              boxes_ref, keep_ref, iou_scr):
    a0 = anch_ref[0:1, :]
    a1 = anch_ref[1:2, :]
    a2 = anch_ref[2:3, :]
    a3 = anch_ref[3:4, :]
    w = a2 - a0 + 1.0
    h = a3 - a1 + 1.0
    cx = (a2 + a0) * 0.5
    cy = (a3 + a1) * 0.5
    dx = preds_ref[0:1, :] * 0.1
    dy = preds_ref[1:2, :] * 0.1
    dw = jnp.minimum(preds_ref[2:3, :] * 0.2, _DWH_CAP)
    dh = jnp.minimum(preds_ref[3:4, :] * 0.2, _DWH_CAP)
    pcx = dx * w + cx
    pcy = dy * h + cy
    pw = jnp.exp(dw) * w
    ph = jnp.exp(dh) * h
    x1 = jnp.clip(pcx - 0.5 * (pw - 1.0), 0.0, _IMG_W - 1.0)
    y1 = jnp.clip(pcy - 0.5 * (ph - 1.0), 0.0, _IMG_H - 1.0)
    x2 = jnp.clip(pcx + 0.5 * (pw - 1.0), 0.0, _IMG_W - 1.0)
    y2 = jnp.clip(pcy + 0.5 * (ph - 1.0), 0.0, _IMG_H - 1.0)
    boxes_ref[0:1, :] = x1
    boxes_ref[1:2, :] = y1
    boxes_ref[2:3, :] = x2
    boxes_ref[3:4, :] = y2

    lane = lax.broadcasted_iota(jnp.int32, (1, _PAD_N), 1)
    real = lane < _TOPN
    neg = jnp.float32(-jnp.inf)
    mx = jnp.maximum(
        jnp.maximum(jnp.max(jnp.where(real, x1, neg)),
                    jnp.max(jnp.where(real, x2, neg))),
        jnp.maximum(jnp.max(jnp.where(real, y1, neg)),
                    jnp.max(jnp.where(real, y2, neg))))
    off = labels_ref[...] * (mx + 1.0)
    ox1 = x1 + off
    oy1 = y1 + off
    ox2 = x2 + off
    oy2 = y2 + off

    area = (ox2 - ox1 + 1.0) * (oy2 - oy1 + 1.0)
    cx1 = jnp.transpose(ox1)
    cy1 = jnp.transpose(oy1)
    cx2 = jnp.transpose(ox2)
    cy2 = jnp.transpose(oy2)
    iw = jnp.clip(jnp.minimum(cx2, ox2) - jnp.maximum(cx1, ox1) + 1.0, 0.0, None)
    ih = jnp.clip(jnp.minimum(cy2, oy2) - jnp.maximum(cy1, oy1) + 1.0, 0.0, None)
    inter = iw * ih
    iou_scr[...] = inter / (jnp.transpose(area) + area - inter)

    nvalid = jnp.sum(valid_ref[...]).astype(jnp.int32)

    def body(i, keep):
        row = iou_scr[pl.ds(i, 1), :]
        lower = lane < i
        sup = jnp.sum(jnp.where(lower & (row > _NMS_T), keep, 0.0)) > 0.0
        newv = jnp.where((i < nvalid) & jnp.logical_not(sup), 1.0, 0.0)
        return jnp.where(lane == i, newv, keep)

    keep = lax.fori_loop(0, _TOPN, body, jnp.zeros((1, _PAD_N), jnp.float32))
    keep_ref[...] = keep


def _vote_body(resb_ref, rlab_ref, rval_ref,
               boxes_ref, labels_ref, cscore_ref,
               voted_ref, den_ref):
    bx1 = boxes_ref[0:1, :]
    by1 = boxes_ref[1:2, :]
    bx2 = boxes_ref[2:3, :]
    by2 = boxes_ref[3:4, :]
    rx1 = jnp.transpose(resb_ref[0:1, :])
    ry1 = jnp.transpose(resb_ref[1:2, :])
    rx2 = jnp.transpose(resb_ref[2:3, :])
    ry2 = jnp.transpose(resb_ref[3:4, :])
    area_c = (bx2 - bx1 + 1.0) * (by2 - by1 + 1.0)
    area_r = (rx2 - rx1 + 1.0) * (ry2 - ry1 + 1.0)
    iw = jnp.clip(jnp.minimum(rx2, bx2) - jnp.maximum(rx1, bx1) + 1.0, 0.0, None)
    ih = jnp.clip(jnp.minimum(ry2, by2) - jnp.maximum(ry1, by1) + 1.0, 0.0, None)
    inter = iw * ih
    iou = inter / (area_r + area_c - inter)
    m = ((iou > 0.01)
         & (jnp.transpose(rlab_ref[...]) == labels_ref[...])
         & (jnp.transpose(rval_ref[...]) > 0.0))
    pis = jnp.exp(-(1.0 - iou) ** 2 / _SIGMA) * cscore_ref[...]
    pis = jnp.where(m, pis, 0.0)
    den = jnp.sum(pis, axis=1, keepdims=True)
    dsafe = jnp.where(den > 0.0, den, 1.0)
    vx1 = jnp.sum(pis * bx1, axis=1, keepdims=True) / dsafe
    vy1 = jnp.sum(pis * by1, axis=1, keepdims=True) / dsafe
    vx2 = jnp.sum(pis * bx2, axis=1, keepdims=True) / dsafe
    vy2 = jnp.sum(pis * by2, axis=1, keepdims=True) / dsafe
    voted_ref[...] = jnp.concatenate([vx1, vy1, vx2, vy2], axis=1)
    den_ref[...] = den


def kernel(box_cls, box_regression, iou_pred, anchors):
    cls2 = box_cls.reshape(_C, _HW)
    iou2 = iou_pred.reshape(1, _HW)

    blk = 2048
    grid = (_HW + blk - 1) // blk
    s_map = pl.pallas_call(
        _score_body,
        grid=(grid,),
        in_specs=[
            pl.BlockSpec((_C, blk), lambda i: (0, i)),
            pl.BlockSpec((1, blk), lambda i: (0, i)),
        ],
        out_specs=pl.BlockSpec((_C, blk), lambda i: (0, i)),
        out_shape=jax.ShapeDtypeStruct((_C, _HW), jnp.float32),
    )(cls2, iou2)

    top_scores, top_idx = lax.top_k(s_map.reshape(-1), _TOPN)
    loc = top_idx % _HW
    labels = top_idx // _HW + 1
    valid = jnp.isfinite(top_scores)

    reg_t = box_regression.reshape(4, _HW)[:, loc]
    anch_t = anchors.astype(jnp.float32).T[:, loc]
    padn = _PAD_N - _TOPN
    reg_p = jnp.pad(reg_t, ((0, 0), (0, padn)))
    anch_p = jnp.pad(anch_t, ((0, 0), (0, padn)))
    lab_p = jnp.pad(labels.astype(jnp.float32)[None, :], ((0, 0), (0, padn)))
    val_p = jnp.pad(valid.astype(jnp.float32)[None, :], ((0, 0), (0, padn)))

    boxes_t, keep_f = pl.pallas_call(
        _nms_body,
        out_shape=(
            jax.ShapeDtypeStruct((4, _PAD_N), jnp.float32),
            jax.ShapeDtypeStruct((1, _PAD_N), jnp.float32),
        ),
        scratch_shapes=[pltpu.VMEM((_PAD_N, _PAD_N), jnp.float32)],
    )(reg_p, anch_p, lab_p, val_p)

    keep = keep_f[0, :_TOPN] > 0.0
    s_kept = jnp.where(keep, top_scores, -jnp.inf)
    res_scores, res_idx = lax.top_k(s_kept, _POST)
    res_valid = jnp.isfinite(res_scores)
    res_labels = labels[res_idx]

    padp = _PAD_P - _POST
    resb_p = jnp.pad(boxes_t[:, res_idx], ((0, 0), (0, padp)))
    rlab_p = jnp.pad(res_labels.astype(jnp.float32)[None, :], ((0, 0), (0, padp)))
    rval_p = jnp.pad(res_valid.astype(jnp.float32)[None, :], ((0, 0), (0, padp)))
    cscore = jnp.where(valid, top_scores, 0.0)
    csc_p = jnp.pad(cscore[None, :], ((0, 0), (0, padn)))

    voted, den = pl.pallas_call(
        _vote_body,
        out_shape=(
            jax.ShapeDtypeStruct((_PAD_P, 4), jnp.float32),
            jax.ShapeDtypeStruct((_PAD_P, 1), jnp.float32),
        ),
    )(resb_p, rlab_p, rval_p, boxes_t, lab_p, csc_p)

    den100 = den[:_POST, 0]
    boxes100 = boxes_t[:, res_idx].T
    res_boxes = jnp.where((den100 > 0.0)[:, None], voted[:_POST], boxes100)
    out_scores = jnp.where(res_valid, res_scores, 0.0)
    out5 = jnp.concatenate([res_boxes, out_scores[:, None]], axis=1)
    return out5, res_labels
